# baseline (speedup 1.0000x reference)
import functools

import jax
import jax.numpy as jnp
import numpy as np
from jax import lax
from jax.experimental import pallas as pl
from jax.experimental.pallas import tpu as pltpu
from jax.experimental.pallas import tpu_sc as plsc

N = 10000
E = 320000
D = 128

NC = 2
NS = 16
NW = NC * NS
CH = 32
TOT = 320
EPT = CH * TOT
E_PAD = NW * EPT
N_PAD = 10240
ROWS_PT = N_PAD // NS
NB = 5
PFG = 3
PFI = 4

_PERM = np.array([32 * (p // 32) + 2 * (p % 16) + ((p % 32) // 16)
                  for p in range(D)], dtype=np.int32)

_mesh = plsc.VectorSubcoreMesh(core_axis_name="c", subcore_axis_name="s")

_BCAST_DNUMS = lax.GatherDimensionNumbers(
    offset_dims=(), collapsed_slice_dims=(0,), start_index_map=(0,))


@functools.partial(
    pl.kernel,
    out_type=jax.ShapeDtypeStruct((2, NC, N_PAD, D), jnp.float32),
    mesh=_mesh,
    compiler_params=pltpu.CompilerParams(needs_layout_passes=False, use_tc_tiling_on_sc=False),
    scratch_types=[
        pltpu.VMEM((NB, CH), jnp.int32),
        pltpu.VMEM((NB, CH), jnp.int32),
        pltpu.VMEM((NB, CH), jnp.float32),
        pltpu.VMEM((NB, CH, D // 2), jnp.int32),
        pltpu.VMEM((NB, CH, D), jnp.float32),
        pltpu.VMEM_SHARED((N_PAD, D), jnp.float32),
        pltpu.SemaphoreType.DMA((NB,)),
        pltpu.SemaphoreType.DMA((NB,)),
        pltpu.SemaphoreType.DMA((NB,)),
        pltpu.SemaphoreType.DMA((NB,)),
        pltpu.SemaphoreType.DMA((NB,)),
    ],
)
def _sc_spmm(x_hbm, row_hbm, col_hbm, val_hbm, out_hbm,
             colb, rowb, valb, rbf, rf32, agg_sh, gsem, ssem, csem, rsem,
             vsem):
    cid = lax.axis_index("c")
    sid = lax.axis_index("s")
    wid = sid * NC + cid
    r0 = sid * ROWS_PT

    zeros16 = jnp.zeros((16,), jnp.float32)

    def _zero_buf0(i, carry):
        for c8 in range(D // 16):
            rf32[0, i, pl.ds(c8 * 16, 16)] = zeros16
        return carry

    def _scale(b):
        def _g(g, carry):
            vals_g = valb[b, pl.ds(g * 16, 16)]
            for e16 in range(16):
                vv = lax.gather(
                    vals_g, jnp.full((16, 1), e16, jnp.int32),
                    _BCAST_DNUMS, (1,),
                    mode=lax.GatherScatterMode.PROMISE_IN_BOUNDS)
                e = g * 16 + e16
                for c in range(D // 32):
                    w = rbf[b, e, pl.ds(c * 16, 16)]
                    lo = plsc.bitcast(w << 16, jnp.float32)
                    hi = plsc.bitcast(
                        w & jnp.int32(-65536), jnp.float32)
                    rf32[b, e, pl.ds(c * 32, 16)] = lo * vv
                    rf32[b, e, pl.ds(c * 32 + 16, 16)] = hi * vv
            return carry

        lax.fori_loop(0, CH // 16, _g, 0)

    def _idx_load(jc, slot):
        pltpu.async_copy(col_hbm.at[wid, jc], colb.at[slot], csem.at[slot])
        pltpu.async_copy(row_hbm.at[wid, jc], rowb.at[slot], rsem.at[slot])
        pltpu.async_copy(val_hbm.at[wid, jc], valb.at[slot], vsem.at[slot])

    def _input(inp, carry):
        xoff = jnp.broadcast_to((inp * N).astype(jnp.int32), (16,))

        def _adjust_col(slot):
            for g in range(CH // 16):
                colb[slot, pl.ds(g * 16, 16)] = (
                    colb[slot, pl.ds(g * 16, 16)] + xoff)

        lax.fori_loop(0, CH, _zero_buf0, 0)
        for k in range(ROWS_PT // CH):
            pltpu.sync_copy(rf32.at[0], agg_sh.at[pl.ds(r0 + k * CH, CH)])
        plsc.subcore_barrier()

        for k in range(PFI):
            _idx_load(k, k)
        for k in range(PFG):
            pltpu.make_async_copy(
                col_hbm.at[wid, k], colb.at[k], csem.at[k]).wait()
            _adjust_col(k)
            pltpu.async_copy(x_hbm.at[colb.at[k]], rbf.at[k], gsem.at[k])

        def _slot(j5, b, carry2):
            j = j5 * NB + b
            b1 = (b + PFI) % NB
            b2 = (b + PFG) % NB
            pltpu.make_async_copy(
                x_hbm.at[colb.at[b]], rbf.at[b], gsem.at[b]).wait()
            pltpu.make_async_copy(
                row_hbm.at[wid, j], rowb.at[b], rsem.at[b]).wait()
            pltpu.make_async_copy(
                val_hbm.at[wid, j], valb.at[b], vsem.at[b]).wait()
            _scale(b)
            pltpu.async_copy(
                rf32.at[b], agg_sh.at[rowb.at[b]], ssem.at[b], add=True)

            @pl.when(j >= 1)
            def _():
                pltpu.make_async_copy(
                    rf32.at[b1], agg_sh.at[rowb.at[b1]],
                    ssem.at[b1]).wait()

            @pl.when(j + PFI < TOT)
            def _():
                _idx_load(j + PFI, b1)

            @pl.when(j + PFG < TOT)
            def _():
                pltpu.make_async_copy(
                    col_hbm.at[wid, j + PFG], colb.at[b2],
                    csem.at[b2]).wait()
                _adjust_col(b2)
                pltpu.async_copy(
                    x_hbm.at[colb.at[b2]], rbf.at[b2], gsem.at[b2])
            return carry2

        def _group(j5, carry2):
            for b in range(NB):
                _slot(j5, b, carry2)
            return carry2

        lax.fori_loop(0, TOT // NB, _group, 0)
        pltpu.make_async_copy(
            rf32.at[(TOT - 1) % NB], agg_sh.at[rowb.at[(TOT - 1) % NB]],
            ssem.at[(TOT - 1) % NB]).wait()
        plsc.subcore_barrier()

        pltpu.sync_copy(agg_sh.at[pl.ds(r0, ROWS_PT)],
                        out_hbm.at[inp, cid, pl.ds(r0, ROWS_PT)])
        plsc.subcore_barrier()
        return carry

    lax.fori_loop(0, 2, _input, 0)


def _epilogue_body(parts_ref, W_ref, bias_ref, prelu_ref, linW_ref, linb_ref,
                   out_ref):
    agg = parts_ref[0, 0] + parts_ref[0, 1]
    h = jnp.dot(agg, W_ref[...].T, preferred_element_type=jnp.float32)
    t = h + bias_ref[0][None, :]
    p = prelu_ref[0, 0]
    t = jnp.where(t >= 0, t, p * t)
    wsum = jnp.sum(linW_ref[...], axis=0)
    bsum = jnp.sum(linb_ref[0])
    z = jnp.sum(t * wsum[None, :], axis=1) + bsum
    out_ref[0, 0, :] = z


def _epilogue(parts, W_fc, gcn_bias, prelu_w, lin_W, lin_b):
    return pl.pallas_call(
        _epilogue_body,
        grid=(2,),
        in_specs=[
            pl.BlockSpec((1, NC, N_PAD, D), lambda i: (i, 0, 0, 0)),
            pl.BlockSpec((D, D), lambda i: (0, 0)),
            pl.BlockSpec((1, D), lambda i: (0, 0)),
            pl.BlockSpec((1, 1), lambda i: (0, 0), memory_space=pltpu.SMEM),
            pl.BlockSpec((D, D), lambda i: (0, 0)),
            pl.BlockSpec((1, D), lambda i: (0, 0)),
        ],
        out_specs=pl.BlockSpec((1, 1, N_PAD), lambda i: (i, 0, 0)),
        out_shape=jax.ShapeDtypeStruct((2, 1, N_PAD), jnp.float32),
    )(parts, W_fc, gcn_bias, prelu_w, lin_W, lin_b)


def kernel(x_1, x_2, adj_indices, adj_values, W_fc, prelu_w, gcn_bias, lin_W,
           lin_b):
    row = adj_indices[0]
    col = adj_indices[1]
    pad = E_PAD - E
    spread = (jnp.arange(pad, dtype=jnp.int32) * 37) % N
    row_p = jnp.concatenate([row, spread])
    col_p = jnp.concatenate([col, spread])
    val_p = jnp.concatenate([adj_values, jnp.zeros((pad,), jnp.float32)])
    xb = jnp.concatenate([x_1, x_2], axis=0).astype(jnp.bfloat16)
    xs = jax.lax.bitcast_convert_type(
        xb.reshape(2 * N, D // 2, 2), jnp.int32)
    parts = _sc_spmm(xs,
                     row_p.reshape(NW, TOT, CH),
                     col_p.reshape(NW, TOT, CH),
                     val_p.reshape(NW, TOT, CH))
    W_perm = W_fc[:, _PERM]
    z = _epilogue(parts, W_perm, gcn_bias.reshape(1, D),
                  prelu_w.reshape(1, 1), lin_W, lin_b.reshape(1, D))
    return z[:, 0, :N].reshape(2 * N)


# speedup vs baseline: 2.3955x; 2.3955x over previous
import functools

import jax
import jax.numpy as jnp
from jax import lax
from jax.experimental import pallas as pl
from jax.experimental.pallas import tpu as pltpu
from jax.experimental.pallas import tpu_sc as plsc

N = 10000
E = 320000
D = 128

NC = 2
NS = 16
NW = NC * NS
CH = 32
TOT = 320
EPT = CH * TOT
E_PAD = NW * EPT
N_PAD = 10240
ROWS_PT = N_PAD // NS
NB = 5
PFG = 3
PFI = 4

_mesh = plsc.VectorSubcoreMesh(core_axis_name="c", subcore_axis_name="s")

_BCAST_DNUMS = lax.GatherDimensionNumbers(
    offset_dims=(), collapsed_slice_dims=(0,), start_index_map=(0,))


@functools.partial(
    pl.kernel,
    out_type=jax.ShapeDtypeStruct((2, NC, N_PAD, D), jnp.float32),
    mesh=_mesh,
    scratch_types=[
        pltpu.VMEM((NB, CH), jnp.int32),
        pltpu.VMEM((NB, CH), jnp.int32),
        pltpu.VMEM((NB, CH), jnp.float32),
        pltpu.VMEM((NB, CH, D), jnp.float32),
        pltpu.VMEM_SHARED((N_PAD, D), jnp.float32),
        pltpu.SemaphoreType.DMA((NB,)),
        pltpu.SemaphoreType.DMA((NB,)),
        pltpu.SemaphoreType.DMA((NB,)),
        pltpu.SemaphoreType.DMA((NB,)),
        pltpu.SemaphoreType.DMA((NB,)),
    ],
)
def _sc_spmm(x1_hbm, x2_hbm, row_hbm, col_hbm, val_hbm, out_hbm,
             colb, rowb, valb, rows4, agg_sh, gsem, ssem, csem, rsem, vsem):
    cid = lax.axis_index("c")
    sid = lax.axis_index("s")
    wid = sid * NC + cid
    r0 = sid * ROWS_PT

    zeros16 = jnp.zeros((16,), jnp.float32)

    def _zero_buf0(i, carry):
        for c8 in range(D // 16):
            rows4[0, i, pl.ds(c8 * 16, 16)] = zeros16
        return carry

    def _scale(b):
        def _g(g, carry):
            vals_g = valb[b, pl.ds(g * 16, 16)]
            for e16 in range(16):
                vv = lax.gather(
                    vals_g, jnp.full((16, 1), e16, jnp.int32),
                    _BCAST_DNUMS, (1,),
                    mode=lax.GatherScatterMode.PROMISE_IN_BOUNDS)
                e = g * 16 + e16
                for c8 in range(D // 16):
                    sl = rows4[b, e, pl.ds(c8 * 16, 16)]
                    rows4[b, e, pl.ds(c8 * 16, 16)] = sl * vv
            return carry

        lax.fori_loop(0, CH // 16, _g, 0)

    def _idx_load(jc, slot):
        pltpu.async_copy(col_hbm.at[wid, jc], colb.at[slot], csem.at[slot])
        pltpu.async_copy(row_hbm.at[wid, jc], rowb.at[slot], rsem.at[slot])
        pltpu.async_copy(val_hbm.at[wid, jc], valb.at[slot], vsem.at[slot])

    def _pipeline(x_hbm):
        for k in range(PFI):
            _idx_load(k, k)
        for k in range(PFG):
            pltpu.make_async_copy(
                col_hbm.at[wid, k], colb.at[k], csem.at[k]).wait()
            pltpu.async_copy(x_hbm.at[colb.at[k]], rows4.at[k], gsem.at[k])

        def _slot(j5, b, carry):
            j = j5 * NB + b
            b1 = (b + PFI) % NB
            b2 = (b + PFG) % NB
            pltpu.make_async_copy(
                x_hbm.at[colb.at[b]], rows4.at[b], gsem.at[b]).wait()
            pltpu.make_async_copy(
                row_hbm.at[wid, j], rowb.at[b], rsem.at[b]).wait()
            pltpu.make_async_copy(
                val_hbm.at[wid, j], valb.at[b], vsem.at[b]).wait()
            _scale(b)
            pltpu.async_copy(
                rows4.at[b], agg_sh.at[rowb.at[b]], ssem.at[b], add=True)

            @pl.when(j >= 1)
            def _():
                pltpu.make_async_copy(
                    rows4.at[b1], agg_sh.at[rowb.at[b1]],
                    ssem.at[b1]).wait()

            @pl.when(j + PFI < TOT)
            def _():
                _idx_load(j + PFI, b1)

            @pl.when(j + PFG < TOT)
            def _():
                pltpu.make_async_copy(
                    col_hbm.at[wid, j + PFG], colb.at[b2],
                    csem.at[b2]).wait()
                pltpu.async_copy(
                    x_hbm.at[colb.at[b2]], rows4.at[b2], gsem.at[b2])
            return carry

        def _group(j5, carry):
            for b in range(NB):
                _slot(j5, b, carry)
            return carry

        lax.fori_loop(0, TOT // NB, _group, 0)
        pltpu.make_async_copy(
            rows4.at[(TOT - 1) % NB], agg_sh.at[rowb.at[(TOT - 1) % NB]],
            ssem.at[(TOT - 1) % NB]).wait()

    for inp, x_hbm in enumerate((x1_hbm, x2_hbm)):
        lax.fori_loop(0, CH, _zero_buf0, 0)
        for k in range(ROWS_PT // CH):
            pltpu.sync_copy(rows4.at[0], agg_sh.at[pl.ds(r0 + k * CH, CH)])
        plsc.subcore_barrier()

        _pipeline(x_hbm)
        plsc.subcore_barrier()

        pltpu.sync_copy(agg_sh.at[pl.ds(r0, ROWS_PT)],
                        out_hbm.at[inp, cid, pl.ds(r0, ROWS_PT)])
        plsc.subcore_barrier()


def _epilogue_body(parts_ref, W_ref, bias_ref, prelu_ref, linW_ref, linb_ref,
                   out_ref):
    agg = parts_ref[0, 0] + parts_ref[0, 1]
    h = jnp.dot(agg, W_ref[...].T, preferred_element_type=jnp.float32)
    t = h + bias_ref[0][None, :]
    p = prelu_ref[0, 0]
    t = jnp.where(t >= 0, t, p * t)
    wsum = jnp.sum(linW_ref[...], axis=0)
    bsum = jnp.sum(linb_ref[0])
    z = jnp.sum(t * wsum[None, :], axis=1) + bsum
    out_ref[0, 0, :] = z


def _epilogue(parts, W_fc, gcn_bias, prelu_w, lin_W, lin_b):
    return pl.pallas_call(
        _epilogue_body,
        grid=(2,),
        in_specs=[
            pl.BlockSpec((1, NC, N_PAD, D), lambda i: (i, 0, 0, 0)),
            pl.BlockSpec((D, D), lambda i: (0, 0)),
            pl.BlockSpec((1, D), lambda i: (0, 0)),
            pl.BlockSpec((1, 1), lambda i: (0, 0), memory_space=pltpu.SMEM),
            pl.BlockSpec((D, D), lambda i: (0, 0)),
            pl.BlockSpec((1, D), lambda i: (0, 0)),
        ],
        out_specs=pl.BlockSpec((1, 1, N_PAD), lambda i: (i, 0, 0)),
        out_shape=jax.ShapeDtypeStruct((2, 1, N_PAD), jnp.float32),
    )(parts, W_fc, gcn_bias, prelu_w, lin_W, lin_b)


def kernel(x_1, x_2, adj_indices, adj_values, W_fc, prelu_w, gcn_bias, lin_W,
           lin_b):
    row = adj_indices[0]
    col = adj_indices[1]
    pad = E_PAD - E
    spread = (jnp.arange(pad, dtype=jnp.int32) * 37) % N
    row_p = jnp.concatenate([row, spread])
    col_p = jnp.concatenate([col, spread])
    val_p = jnp.concatenate([adj_values, jnp.zeros((pad,), jnp.float32)])
    parts = _sc_spmm(x_1, x_2,
                     row_p.reshape(NW, TOT, CH),
                     col_p.reshape(NW, TOT, CH),
                     val_p.reshape(NW, TOT, CH))
    z = _epilogue(parts, W_fc, gcn_bias.reshape(1, D),
                  prelu_w.reshape(1, 1), lin_W, lin_b.reshape(1, D))
    return z[:, 0, :N].reshape(2 * N)


# speedup vs baseline: 2.7732x; 1.1577x over previous
import functools

import jax
import jax.numpy as jnp
from jax import lax
from jax.experimental import pallas as pl
from jax.experimental.pallas import tpu as pltpu
from jax.experimental.pallas import tpu_sc as plsc

N = 10000
E = 320000
D = 128

NC = 2
NS = 16
NW = NC * NS
CH = 80
TOT = 128
EPT = CH * TOT
E_PAD = NW * EPT
N_PAD = 10240
ROWS_PT = N_PAD // NS
NB = 4
PFG = 2
PFI = 3

_mesh = plsc.VectorSubcoreMesh(core_axis_name="c", subcore_axis_name="s")

_BCAST_DNUMS = lax.GatherDimensionNumbers(
    offset_dims=(), collapsed_slice_dims=(0,), start_index_map=(0,))


@functools.partial(
    pl.kernel,
    out_type=jax.ShapeDtypeStruct((2, NC, N_PAD, D), jnp.float32),
    mesh=_mesh,
    scratch_types=[
        pltpu.VMEM((NB, CH), jnp.int32),
        pltpu.VMEM((NB, CH), jnp.int32),
        pltpu.VMEM((NB, CH), jnp.float32),
        pltpu.VMEM((NB, CH, D), jnp.float32),
        pltpu.VMEM_SHARED((N_PAD, D), jnp.float32),
        pltpu.SemaphoreType.DMA((NB,)),
        pltpu.SemaphoreType.DMA((NB,)),
        pltpu.SemaphoreType.DMA((NB,)),
        pltpu.SemaphoreType.DMA((NB,)),
        pltpu.SemaphoreType.DMA((NB,)),
    ],
)
def _sc_spmm(x1_hbm, x2_hbm, row_hbm, col_hbm, val_hbm, out_hbm,
             colb, rowb, valb, rows4, agg_sh, gsem, ssem, csem, rsem, vsem):
    cid = lax.axis_index("c")
    sid = lax.axis_index("s")
    wid = sid * NC + cid
    r0 = sid * ROWS_PT

    zeros16 = jnp.zeros((16,), jnp.float32)

    def _zero_buf0(i, carry):
        for c8 in range(D // 16):
            rows4[0, i, pl.ds(c8 * 16, 16)] = zeros16
        return carry

    def _scale(b):
        def _g(g, carry):
            vals_g = valb[b, pl.ds(g * 16, 16)]
            for e16 in range(16):
                vv = lax.gather(
                    vals_g, jnp.full((16, 1), e16, jnp.int32),
                    _BCAST_DNUMS, (1,),
                    mode=lax.GatherScatterMode.PROMISE_IN_BOUNDS)
                e = g * 16 + e16
                for c8 in range(D // 16):
                    sl = rows4[b, e, pl.ds(c8 * 16, 16)]
                    rows4[b, e, pl.ds(c8 * 16, 16)] = sl * vv
            return carry

        lax.fori_loop(0, CH // 16, _g, 0)

    def _idx_load(jc, slot):
        pltpu.async_copy(col_hbm.at[wid, jc], colb.at[slot], csem.at[slot])
        pltpu.async_copy(row_hbm.at[wid, jc], rowb.at[slot], rsem.at[slot])
        pltpu.async_copy(val_hbm.at[wid, jc], valb.at[slot], vsem.at[slot])

    def _pipeline(x_hbm):
        for k in range(PFI):
            _idx_load(k, k)
        for k in range(PFG):
            pltpu.make_async_copy(
                col_hbm.at[wid, k], colb.at[k], csem.at[k]).wait()
            pltpu.async_copy(x_hbm.at[colb.at[k]], rows4.at[k], gsem.at[k])

        def _slot(j5, b, carry):
            j = j5 * NB + b
            b1 = (b + PFI) % NB
            b2 = (b + PFG) % NB
            pltpu.make_async_copy(
                x_hbm.at[colb.at[b]], rows4.at[b], gsem.at[b]).wait()
            pltpu.make_async_copy(
                row_hbm.at[wid, j], rowb.at[b], rsem.at[b]).wait()
            pltpu.make_async_copy(
                val_hbm.at[wid, j], valb.at[b], vsem.at[b]).wait()
            _scale(b)
            pltpu.async_copy(
                rows4.at[b], agg_sh.at[rowb.at[b]], ssem.at[b], add=True)

            @pl.when(j >= 1)
            def _():
                pltpu.make_async_copy(
                    rows4.at[b1], agg_sh.at[rowb.at[b1]],
                    ssem.at[b1]).wait()

            @pl.when(j + PFI < TOT)
            def _():
                _idx_load(j + PFI, b1)

            @pl.when(j + PFG < TOT)
            def _():
                pltpu.make_async_copy(
                    col_hbm.at[wid, j + PFG], colb.at[b2],
                    csem.at[b2]).wait()
                pltpu.async_copy(
                    x_hbm.at[colb.at[b2]], rows4.at[b2], gsem.at[b2])
            return carry

        def _group(j5, carry):
            for b in range(NB):
                _slot(j5, b, carry)
            return carry

        lax.fori_loop(0, TOT // NB, _group, 0)
        pltpu.make_async_copy(
            rows4.at[(TOT - 1) % NB], agg_sh.at[rowb.at[(TOT - 1) % NB]],
            ssem.at[(TOT - 1) % NB]).wait()

    for inp, x_hbm in enumerate((x1_hbm, x2_hbm)):
        lax.fori_loop(0, CH, _zero_buf0, 0)
        for k in range(ROWS_PT // CH):
            pltpu.sync_copy(rows4.at[0], agg_sh.at[pl.ds(r0 + k * CH, CH)])
        plsc.subcore_barrier()

        _pipeline(x_hbm)
        plsc.subcore_barrier()

        pltpu.sync_copy(agg_sh.at[pl.ds(r0, ROWS_PT)],
                        out_hbm.at[inp, cid, pl.ds(r0, ROWS_PT)])
        plsc.subcore_barrier()


def _epilogue_body(parts_ref, W_ref, bias_ref, prelu_ref, linW_ref, linb_ref,
                   out_ref):
    agg = parts_ref[0, 0] + parts_ref[0, 1]
    h = jnp.dot(agg, W_ref[...].T, preferred_element_type=jnp.float32)
    t = h + bias_ref[0][None, :]
    p = prelu_ref[0, 0]
    t = jnp.where(t >= 0, t, p * t)
    wsum = jnp.sum(linW_ref[...], axis=0)
    bsum = jnp.sum(linb_ref[0])
    z = jnp.sum(t * wsum[None, :], axis=1) + bsum
    out_ref[0, 0, :] = z


def _epilogue(parts, W_fc, gcn_bias, prelu_w, lin_W, lin_b):
    return pl.pallas_call(
        _epilogue_body,
        grid=(2,),
        in_specs=[
            pl.BlockSpec((1, NC, N_PAD, D), lambda i: (i, 0, 0, 0)),
            pl.BlockSpec((D, D), lambda i: (0, 0)),
            pl.BlockSpec((1, D), lambda i: (0, 0)),
            pl.BlockSpec((1, 1), lambda i: (0, 0), memory_space=pltpu.SMEM),
            pl.BlockSpec((D, D), lambda i: (0, 0)),
            pl.BlockSpec((1, D), lambda i: (0, 0)),
        ],
        out_specs=pl.BlockSpec((1, 1, N_PAD), lambda i: (i, 0, 0)),
        out_shape=jax.ShapeDtypeStruct((2, 1, N_PAD), jnp.float32),
    )(parts, W_fc, gcn_bias, prelu_w, lin_W, lin_b)


def kernel(x_1, x_2, adj_indices, adj_values, W_fc, prelu_w, gcn_bias, lin_W,
           lin_b):
    row = adj_indices[0]
    col = adj_indices[1]
    pad = E_PAD - E
    spread = (jnp.arange(pad, dtype=jnp.int32) * 37) % N
    row_p = jnp.concatenate([row, spread])
    col_p = jnp.concatenate([col, spread])
    val_p = jnp.concatenate([adj_values, jnp.zeros((pad,), jnp.float32)])
    parts = _sc_spmm(x_1, x_2,
                     row_p.reshape(NW, TOT, CH),
                     col_p.reshape(NW, TOT, CH),
                     val_p.reshape(NW, TOT, CH))
    z = _epilogue(parts, W_fc, gcn_bias.reshape(1, D),
                  prelu_w.reshape(1, 1), lin_W, lin_b.reshape(1, D))
    return z[:, 0, :N].reshape(2 * N)


# speedup vs baseline: 2.9314x; 1.0570x over previous
import functools

import jax
import jax.numpy as jnp
import numpy as np
from jax import lax
from jax.experimental import pallas as pl
from jax.experimental.pallas import tpu as pltpu
from jax.experimental.pallas import tpu_sc as plsc

N = 10000
E = 320000
D = 128

NC = 2
NS = 16
NW = NC * NS
CH = 64
TOT = 160
EPT = CH * TOT
E_PAD = NW * EPT
N_PAD = 10240
ROWS_PT = N_PAD // NS
NB = 5
PFG = 3
PFI = 4

_mesh = plsc.VectorSubcoreMesh(core_axis_name="c", subcore_axis_name="s")

_BCAST_DNUMS = lax.GatherDimensionNumbers(
    offset_dims=(), collapsed_slice_dims=(0,), start_index_map=(0,))


@functools.partial(
    pl.kernel,
    out_type=jax.ShapeDtypeStruct((2, NC, N_PAD, D), jnp.float32),
    mesh=_mesh,
    scratch_types=[
        pltpu.VMEM((NB, CH), jnp.int32),
        pltpu.VMEM((NB, CH), jnp.int32),
        pltpu.VMEM((NB, CH), jnp.float32),
        pltpu.VMEM((NB, CH, D), jnp.float32),
        pltpu.VMEM_SHARED((N_PAD, D), jnp.float32),
        pltpu.SemaphoreType.DMA((NB,)),
        pltpu.SemaphoreType.DMA((NB,)),
        pltpu.SemaphoreType.DMA((NB,)),
        pltpu.SemaphoreType.DMA((NB,)),
        pltpu.SemaphoreType.DMA((NB,)),
    ],
)
def _sc_spmm(x1_hbm, x2_hbm, row_hbm, col_hbm, val_hbm, out_hbm,
             colb, rowb, valb, rows4, agg_sh, gsem, ssem, csem, rsem, vsem):
    cid = lax.axis_index("c")
    sid = lax.axis_index("s")
    wid = sid * NC + cid
    r0 = sid * ROWS_PT

    zeros16 = jnp.zeros((16,), jnp.float32)

    def _zero_buf0(i, carry):
        for c8 in range(D // 16):
            rows4[0, i, pl.ds(c8 * 16, 16)] = zeros16
        return carry

    def _scale(b):
        def _g(g, carry):
            vals_g = valb[b, pl.ds(g * 16, 16)]
            for e16 in range(16):
                vv = lax.gather(
                    vals_g, jnp.full((16, 1), e16, jnp.int32),
                    _BCAST_DNUMS, (1,),
                    mode=lax.GatherScatterMode.PROMISE_IN_BOUNDS)
                e = g * 16 + e16
                for c8 in range(D // 16):
                    sl = rows4[b, e, pl.ds(c8 * 16, 16)]
                    rows4[b, e, pl.ds(c8 * 16, 16)] = sl * vv
            return carry

        lax.fori_loop(0, CH // 16, _g, 0)

    def _idx_load(jc, slot):
        pltpu.async_copy(col_hbm.at[wid, jc], colb.at[slot], csem.at[slot])
        pltpu.async_copy(row_hbm.at[wid, jc], rowb.at[slot], rsem.at[slot])
        pltpu.async_copy(val_hbm.at[wid, jc], valb.at[slot], vsem.at[slot])

    def _pipeline(x_hbm):
        for k in range(PFI):
            _idx_load(k, k)
        for k in range(PFG):
            pltpu.make_async_copy(
                col_hbm.at[wid, k], colb.at[k], csem.at[k]).wait()
            pltpu.async_copy(x_hbm.at[colb.at[k]], rows4.at[k], gsem.at[k])

        def _slot(j5, b, carry):
            j = j5 * NB + b
            b1 = (b + PFI) % NB
            b2 = (b + PFG) % NB
            pltpu.make_async_copy(
                x_hbm.at[colb.at[b]], rows4.at[b], gsem.at[b]).wait()
            pltpu.make_async_copy(
                row_hbm.at[wid, j], rowb.at[b], rsem.at[b]).wait()
            pltpu.make_async_copy(
                val_hbm.at[wid, j], valb.at[b], vsem.at[b]).wait()
            _scale(b)
            pltpu.async_copy(
                rows4.at[b], agg_sh.at[rowb.at[b]], ssem.at[b], add=True)

            @pl.when(j >= 1)
            def _():
                pltpu.make_async_copy(
                    rows4.at[b1], agg_sh.at[rowb.at[b1]],
                    ssem.at[b1]).wait()

            @pl.when(j + PFI < TOT)
            def _():
                _idx_load(j + PFI, b1)

            @pl.when(j + PFG < TOT)
            def _():
                pltpu.make_async_copy(
                    col_hbm.at[wid, j + PFG], colb.at[b2],
                    csem.at[b2]).wait()
                pltpu.async_copy(
                    x_hbm.at[colb.at[b2]], rows4.at[b2], gsem.at[b2])
            return carry

        def _group(j5, carry):
            for b in range(NB):
                _slot(j5, b, carry)
            return carry

        lax.fori_loop(0, TOT // NB, _group, 0)
        pltpu.make_async_copy(
            rows4.at[(TOT - 1) % NB], agg_sh.at[rowb.at[(TOT - 1) % NB]],
            ssem.at[(TOT - 1) % NB]).wait()

    def _zero_agg():
        lax.fori_loop(0, CH, _zero_buf0, 0)
        for k in range(ROWS_PT // CH):
            pltpu.async_copy(rows4.at[0],
                             agg_sh.at[pl.ds(r0 + k * CH, CH)],
                             gsem.at[k % NB])
        for k in range(ROWS_PT // CH):
            pltpu.make_async_copy(rows4.at[0],
                                  agg_sh.at[pl.ds(r0 + k * CH, CH)],
                                  gsem.at[k % NB]).wait()

    _zero_agg()
    plsc.subcore_barrier()

    for inp, x_hbm in enumerate((x1_hbm, x2_hbm)):
        _pipeline(x_hbm)
        plsc.subcore_barrier()

        pltpu.sync_copy(agg_sh.at[pl.ds(r0, ROWS_PT)],
                        out_hbm.at[inp, cid, pl.ds(r0, ROWS_PT)])
        if inp == 0:
            _zero_agg()
        plsc.subcore_barrier()


def _epilogue_body(parts_ref, W_ref, bias_ref, prelu_ref, linW_ref, linb_ref,
                   out_ref):
    agg = parts_ref[0, 0] + parts_ref[0, 1]
    h = jnp.dot(agg, W_ref[...].T, preferred_element_type=jnp.float32)
    t = h + bias_ref[0][None, :]
    p = prelu_ref[0, 0]
    t = jnp.where(t >= 0, t, p * t)
    wsum = jnp.sum(linW_ref[...], axis=0)
    bsum = jnp.sum(linb_ref[0])
    z = jnp.sum(t * wsum[None, :], axis=1) + bsum
    out_ref[0, 0, :] = z


def _epilogue(parts, W_fc, gcn_bias, prelu_w, lin_W, lin_b):
    return pl.pallas_call(
        _epilogue_body,
        grid=(2,),
        in_specs=[
            pl.BlockSpec((1, NC, N_PAD, D), lambda i: (i, 0, 0, 0)),
            pl.BlockSpec((D, D), lambda i: (0, 0)),
            pl.BlockSpec((1, D), lambda i: (0, 0)),
            pl.BlockSpec((1, 1), lambda i: (0, 0), memory_space=pltpu.SMEM),
            pl.BlockSpec((D, D), lambda i: (0, 0)),
            pl.BlockSpec((1, D), lambda i: (0, 0)),
        ],
        out_specs=pl.BlockSpec((1, 1, N_PAD), lambda i: (i, 0, 0)),
        out_shape=jax.ShapeDtypeStruct((2, 1, N_PAD), jnp.float32),
    )(parts, W_fc, gcn_bias, prelu_w, lin_W, lin_b)


def kernel(x_1, x_2, adj_indices, adj_values, W_fc, prelu_w, gcn_bias, lin_W,
           lin_b):
    row = adj_indices[0]
    col = adj_indices[1]
    pad = E_PAD - E
    spread = jnp.asarray((np.arange(pad, dtype=np.int32) * 37) % N)
    row_p = jnp.concatenate([row, spread])
    col_p = jnp.concatenate([col, spread])
    val_p = jnp.concatenate([adj_values, jnp.zeros((pad,), jnp.float32)])
    parts = _sc_spmm(x_1, x_2,
                     row_p.reshape(NW, TOT, CH),
                     col_p.reshape(NW, TOT, CH),
                     val_p.reshape(NW, TOT, CH))
    z = _epilogue(parts, W_fc, gcn_bias.reshape(1, D),
                  prelu_w.reshape(1, 1), lin_W, lin_b.reshape(1, D))
    return z[:, 0, :N].reshape(2 * N)
